# baseline (speedup 1.0000x reference)
import functools

import jax
import jax.numpy as jnp
from jax import lax
from jax.experimental import pallas as pl
from jax.experimental.pallas import tpu as pltpu
from jax.experimental.pallas import tpu_sc as plsc

BEAM = 32
_NEG = float("-inf")
NW = 32


def _sc_gather(scores, ii):
    B, P = ii.shape
    RW = B // NW
    V = scores.shape[1]
    NS = 16

    H0 = 50048
    H1 = V - H0

    @functools.partial(
        pl.kernel,
        mesh=plsc.VectorSubcoreMesh(core_axis_name="c", subcore_axis_name="s"),
        out_type=jax.ShapeDtypeStruct((B * V,), jnp.float32),
        scratch_types=[
            pltpu.VMEM((H0,), jnp.float32),
            pltpu.VMEM((H1,), jnp.float32),
            pltpu.SemaphoreType.DMA,
        ],
    )
    def detile(scores_hbm, flat_hbm, buf0, buf1, rsem):
        wid = lax.axis_index("s") * 2 + lax.axis_index("c")
        base = wid * RW
        bufs = [buf0, buf1]
        pieces = [
            (base + r, h * H0, H0 if h == 0 else H1)
            for r in range(RW)
            for h in range(2)
        ]

        def read(p):
            row, off, sz = pieces[p]
            return pltpu.async_copy(
                scores_hbm.at[row].at[pl.ds(off, sz)],
                bufs[p % 2],
                rsem,
            )

        rd = read(0)
        for p in range(len(pieces)):
            rd.wait()
            if p + 1 < len(pieces):
                rd = read(p + 1)
            row, off, sz = pieces[p]
            pltpu.sync_copy(
                bufs[p % 2],
                flat_hbm.at[pl.ds(row * V + off, sz)],
            )

    @functools.partial(
        pl.kernel,
        mesh=plsc.VectorSubcoreMesh(core_axis_name="c", subcore_axis_name="s"),
        out_type=jax.ShapeDtypeStruct((B, P), jnp.float32),
        scratch_types=[
            pltpu.VMEM((RW, P), jnp.int32),
            pltpu.VMEM((RW, P), jnp.float32),
            pltpu.SemaphoreType.DMA,
        ],
    )
    def g(flat_hbm, ids_hbm, tmp_hbm, idx_v, vals_v, sem):
        wid = lax.axis_index("s") * 2 + lax.axis_index("c")
        base = wid * RW
        pltpu.sync_copy(ids_hbm.at[pl.ds(base, RW)], idx_v)
        for r in range(RW):
            off = (base + r) * V
            for c in range(P // 16):
                sl = (r, pl.ds(c * 16, 16))
                idx_v[sl] = idx_v[sl] + off
        cps = [
            pltpu.async_copy(flat_hbm.at[idx_v.at[r]], vals_v.at[r], sem)
            for r in range(RW)
        ]
        for cp in cps:
            cp.wait()
        pltpu.sync_copy(vals_v, tmp_hbm.at[pl.ds(base, RW)])

    return g(detile(scores), ii)


def _topk_body(tmp_ref, ids_ref, tv_ref, ti_ref, lv_ref, li_ref):
    v = tmp_ref[...]
    ii = ids_ref[...]
    B, P = v.shape
    kk = lax.broadcasted_iota(jnp.int32, (B, P, P), 1)
    jj = lax.broadcasted_iota(jnp.int32, (B, P, P), 2)
    idk = ii[:, :, None]
    idj = ii[:, None, :]
    eqid = idk == idj
    dup = jnp.any(eqid & (kk < jj), axis=1)
    vm = jnp.where(dup, _NEG, v)

    vk = vm[:, :, None]
    vj = vm[:, None, :]
    beats_f = (vk > vj) | ((vk == vj) & ((idk < idj) | (eqid & (kk < jj))))
    rank_f = jnp.sum(beats_f.astype(jnp.int32), axis=1)

    uk = v[:, :, None]
    uj = v[:, None, :]
    beats_l = (uk > uj) | ((uk == uj) & (kk < jj))
    rank_l = jnp.sum(beats_l.astype(jnp.int32), axis=1)

    pp = lax.broadcasted_iota(jnp.int32, (B, P, BEAM), 2)
    self_j = lax.broadcasted_iota(jnp.int32, (B, P, BEAM), 1)
    dupi = dup.astype(jnp.int32)
    onef = (rank_f[:, :, None] == pp) & (dupi[:, :, None] == 0)
    tv = jnp.sum(jnp.where(onef, vm[:, :, None], 0.0), axis=1)
    ti = jnp.sum(jnp.where(onef, ii[:, :, None], 0), axis=1)

    C = 80
    nvalid = P - jnp.sum(dup.astype(jnp.int32), axis=1, keepdims=True)
    cc = lax.broadcasted_iota(jnp.int32, (B, C), 1)
    present = jnp.any(cc[:, :, None] == ii[:, None, :], axis=2)
    absenti = 1 - present.astype(jnp.int32)
    cck = lax.broadcasted_iota(jnp.int32, (B, C, C), 1)
    ccj = lax.broadcasted_iota(jnp.int32, (B, C, C), 2)
    arank = jnp.sum(jnp.where(cck < ccj, absenti[:, :, None], 0), axis=1)
    slot_c = nvalid + arank
    ppc = lax.broadcasted_iota(jnp.int32, (B, C, BEAM), 2)
    onec = (slot_c[:, :, None] == ppc) & (absenti[:, :, None] == 1)
    ti = ti + jnp.sum(jnp.where(onec, cc[:, :, None], 0), axis=1)
    ppb = lax.broadcasted_iota(jnp.int32, (B, BEAM), 1)
    tv = jnp.where(ppb >= nvalid, _NEG, tv)

    tv_ref[...] = tv
    ti_ref[...] = ti
    onel = rank_l[:, :, None] == pp
    lv_ref[...] = jnp.sum(jnp.where(onel, v[:, :, None], 0.0), axis=1)
    li_ref[...] = jnp.sum(jnp.where(onel, self_j, 0), axis=1)


def _tc_topk(tmp, ii):
    B, _ = tmp.shape
    return pl.pallas_call(
        _topk_body,
        out_shape=(
            jax.ShapeDtypeStruct((B, BEAM), jnp.float32),
            jax.ShapeDtypeStruct((B, BEAM), jnp.int32),
            jax.ShapeDtypeStruct((B, BEAM), jnp.float32),
            jax.ShapeDtypeStruct((B, BEAM), jnp.int32),
        ),
    )(tmp, ii)


def kernel(weighted_scores, ids):
    ii = ids.astype(jnp.int32)
    tmp = _sc_gather(weighted_scores, ii)
    return _tc_topk(tmp, ii)


# speedup vs baseline: 1.0616x; 1.0616x over previous
import functools

import jax
import jax.numpy as jnp
from jax import lax
from jax.experimental import pallas as pl
from jax.experimental.pallas import tpu as pltpu
from jax.experimental.pallas import tpu_sc as plsc

BEAM = 32
_NEG = float("-inf")
NW = 32


def _sc_gather(scores, ii):
    B, P = ii.shape
    RW = B // NW
    V = scores.shape[1]
    NS = 16

    @functools.partial(
        pl.kernel,
        mesh=plsc.VectorSubcoreMesh(core_axis_name="c", subcore_axis_name="s"),
        out_type=jax.ShapeDtypeStruct((B * V,), jnp.float32),
        scratch_types=[
            pltpu.VMEM((V,), jnp.float32),
        ],
    )
    def detile(scores_hbm, flat_hbm, row_v):
        wid = lax.axis_index("s") * 2 + lax.axis_index("c")
        base = wid * RW
        for r in range(RW):
            pltpu.sync_copy(scores_hbm.at[base + r], row_v)
            pltpu.sync_copy(row_v, flat_hbm.at[pl.ds((base + r) * V, V)])

    @functools.partial(
        pl.kernel,
        mesh=plsc.VectorSubcoreMesh(core_axis_name="c", subcore_axis_name="s"),
        out_type=jax.ShapeDtypeStruct((B, P), jnp.float32),
        scratch_types=[
            pltpu.VMEM((RW, P), jnp.int32),
            pltpu.VMEM((RW, P), jnp.float32),
            pltpu.SemaphoreType.DMA,
        ],
    )
    def g(flat_hbm, ids_hbm, tmp_hbm, idx_v, vals_v, sem):
        wid = lax.axis_index("s") * 2 + lax.axis_index("c")
        base = wid * RW
        pltpu.sync_copy(ids_hbm.at[pl.ds(base, RW)], idx_v)
        for r in range(RW):
            off = (base + r) * V
            for c in range(P // 16):
                sl = (r, pl.ds(c * 16, 16))
                idx_v[sl] = idx_v[sl] + off
        cps = [
            pltpu.async_copy(flat_hbm.at[idx_v.at[r]], vals_v.at[r], sem)
            for r in range(RW)
        ]
        for cp in cps:
            cp.wait()
        pltpu.sync_copy(vals_v, tmp_hbm.at[pl.ds(base, RW)])

    return g(detile(scores), ii)


def _topk_body(tmp_ref, ids_ref, tv_ref, ti_ref, lv_ref, li_ref):
    v = tmp_ref[...]
    ii = ids_ref[...]
    B, P = v.shape
    kk = lax.broadcasted_iota(jnp.int32, (B, P, P), 1)
    jj = lax.broadcasted_iota(jnp.int32, (B, P, P), 2)
    idk = ii[:, :, None]
    idj = ii[:, None, :]
    eqid = idk == idj
    dup = jnp.any(eqid & (kk < jj), axis=1)
    vm = jnp.where(dup, _NEG, v)

    vk = vm[:, :, None]
    vj = vm[:, None, :]
    beats_f = (vk > vj) | ((vk == vj) & ((idk < idj) | (eqid & (kk < jj))))
    rank_f = jnp.sum(beats_f.astype(jnp.int32), axis=1)

    uk = v[:, :, None]
    uj = v[:, None, :]
    beats_l = (uk > uj) | ((uk == uj) & (kk < jj))
    rank_l = jnp.sum(beats_l.astype(jnp.int32), axis=1)

    pp = lax.broadcasted_iota(jnp.int32, (B, P, BEAM), 2)
    self_j = lax.broadcasted_iota(jnp.int32, (B, P, BEAM), 1)
    dupi = dup.astype(jnp.int32)
    onef = (rank_f[:, :, None] == pp) & (dupi[:, :, None] == 0)
    tv = jnp.sum(jnp.where(onef, vm[:, :, None], 0.0), axis=1)
    ti = jnp.sum(jnp.where(onef, ii[:, :, None], 0), axis=1)

    tv_ref[...] = tv
    ti_ref[...] = ti

    nvalid = P - jnp.sum(dup.astype(jnp.int32), axis=1, keepdims=True)

    @pl.when(jnp.min(nvalid) < BEAM)
    def _degenerate():
        C = 80
        cc = lax.broadcasted_iota(jnp.int32, (B, C), 1)
        present = jnp.any(cc[:, :, None] == ii[:, None, :], axis=2)
        absenti = 1 - present.astype(jnp.int32)
        cck = lax.broadcasted_iota(jnp.int32, (B, C, C), 1)
        ccj = lax.broadcasted_iota(jnp.int32, (B, C, C), 2)
        arank = jnp.sum(jnp.where(cck < ccj, absenti[:, :, None], 0), axis=1)
        slot_c = nvalid + arank
        ppc = lax.broadcasted_iota(jnp.int32, (B, C, BEAM), 2)
        onec = (slot_c[:, :, None] == ppc) & (absenti[:, :, None] == 1)
        ppb = lax.broadcasted_iota(jnp.int32, (B, BEAM), 1)
        ti_ref[...] = ti + jnp.sum(jnp.where(onec, cc[:, :, None], 0), axis=1)
        tv_ref[...] = jnp.where(ppb >= nvalid, _NEG, tv)
    onel = rank_l[:, :, None] == pp
    lv_ref[...] = jnp.sum(jnp.where(onel, v[:, :, None], 0.0), axis=1)
    li_ref[...] = jnp.sum(jnp.where(onel, self_j, 0), axis=1)


def _tc_topk(tmp, ii):
    B, _ = tmp.shape
    return pl.pallas_call(
        _topk_body,
        out_shape=(
            jax.ShapeDtypeStruct((B, BEAM), jnp.float32),
            jax.ShapeDtypeStruct((B, BEAM), jnp.int32),
            jax.ShapeDtypeStruct((B, BEAM), jnp.float32),
            jax.ShapeDtypeStruct((B, BEAM), jnp.int32),
        ),
    )(tmp, ii)


def kernel(weighted_scores, ids):
    ii = ids.astype(jnp.int32)
    tmp = _sc_gather(weighted_scores, ii)
    return _tc_topk(tmp, ii)


# speedup vs baseline: 1.1021x; 1.0381x over previous
import functools

import jax
import jax.numpy as jnp
from jax import lax
from jax.experimental import pallas as pl
from jax.experimental.pallas import tpu as pltpu
from jax.experimental.pallas import tpu_sc as plsc

BEAM = 32
_NEG = float("-inf")
NW = 32


def _sc_gather(scores, ii):
    B, P = ii.shape
    RW = B // NW
    V = scores.shape[1]
    NS = 16

    def _flatten_body(in_ref, out_ref):
        for k in range(32):
            out_ref[pl.ds(k * V, V)] = in_ref[k]

    def detile(scores_in):
        return pl.pallas_call(
            _flatten_body,
            grid=(B // 32,),
            in_specs=[pl.BlockSpec((32, V), lambda i: (i, 0))],
            out_specs=pl.BlockSpec((32 * V,), lambda i: (i,)),
            out_shape=jax.ShapeDtypeStruct((B * V,), jnp.float32),
        )(scores_in)

    @functools.partial(
        pl.kernel,
        mesh=plsc.VectorSubcoreMesh(core_axis_name="c", subcore_axis_name="s"),
        out_type=jax.ShapeDtypeStruct((B, P), jnp.float32),
        scratch_types=[
            pltpu.VMEM((RW, P), jnp.int32),
            pltpu.VMEM((RW, P), jnp.float32),
            pltpu.SemaphoreType.DMA,
        ],
    )
    def g(flat_hbm, ids_hbm, tmp_hbm, idx_v, vals_v, sem):
        wid = lax.axis_index("s") * 2 + lax.axis_index("c")
        base = wid * RW
        pltpu.sync_copy(ids_hbm.at[pl.ds(base, RW)], idx_v)
        for r in range(RW):
            off = (base + r) * V
            for c in range(P // 16):
                sl = (r, pl.ds(c * 16, 16))
                idx_v[sl] = idx_v[sl] + off
        cps = [
            pltpu.async_copy(flat_hbm.at[idx_v.at[r]], vals_v.at[r], sem)
            for r in range(RW)
        ]
        for cp in cps:
            cp.wait()
        pltpu.sync_copy(vals_v, tmp_hbm.at[pl.ds(base, RW)])

    return g(detile(scores), ii)


def _topk_body(tmp_ref, ids_ref, tv_ref, ti_ref, lv_ref, li_ref):
    v = tmp_ref[...]
    ii = ids_ref[...]
    B, P = v.shape
    kk = lax.broadcasted_iota(jnp.int32, (B, P, P), 1)
    jj = lax.broadcasted_iota(jnp.int32, (B, P, P), 2)
    idk = ii[:, :, None]
    idj = ii[:, None, :]
    eqid = idk == idj
    dup = jnp.any(eqid & (kk < jj), axis=1)
    vm = jnp.where(dup, _NEG, v)

    vk = vm[:, :, None]
    vj = vm[:, None, :]
    beats_f = (vk > vj) | ((vk == vj) & ((idk < idj) | (eqid & (kk < jj))))
    rank_f = jnp.sum(beats_f.astype(jnp.int32), axis=1)

    uk = v[:, :, None]
    uj = v[:, None, :]
    beats_l = (uk > uj) | ((uk == uj) & (kk < jj))
    rank_l = jnp.sum(beats_l.astype(jnp.int32), axis=1)

    pp = lax.broadcasted_iota(jnp.int32, (B, P, BEAM), 2)
    self_j = lax.broadcasted_iota(jnp.int32, (B, P, BEAM), 1)
    dupi = dup.astype(jnp.int32)
    onef = (rank_f[:, :, None] == pp) & (dupi[:, :, None] == 0)
    tv = jnp.sum(jnp.where(onef, vm[:, :, None], 0.0), axis=1)
    ti = jnp.sum(jnp.where(onef, ii[:, :, None], 0), axis=1)

    tv_ref[...] = tv
    ti_ref[...] = ti

    nvalid = P - jnp.sum(dup.astype(jnp.int32), axis=1, keepdims=True)

    @pl.when(jnp.min(nvalid) < BEAM)
    def _degenerate():
        C = 80
        cc = lax.broadcasted_iota(jnp.int32, (B, C), 1)
        present = jnp.any(cc[:, :, None] == ii[:, None, :], axis=2)
        absenti = 1 - present.astype(jnp.int32)
        cck = lax.broadcasted_iota(jnp.int32, (B, C, C), 1)
        ccj = lax.broadcasted_iota(jnp.int32, (B, C, C), 2)
        arank = jnp.sum(jnp.where(cck < ccj, absenti[:, :, None], 0), axis=1)
        slot_c = nvalid + arank
        ppc = lax.broadcasted_iota(jnp.int32, (B, C, BEAM), 2)
        onec = (slot_c[:, :, None] == ppc) & (absenti[:, :, None] == 1)
        ppb = lax.broadcasted_iota(jnp.int32, (B, BEAM), 1)
        ti_ref[...] = ti + jnp.sum(jnp.where(onec, cc[:, :, None], 0), axis=1)
        tv_ref[...] = jnp.where(ppb >= nvalid, _NEG, tv)
    onel = rank_l[:, :, None] == pp
    lv_ref[...] = jnp.sum(jnp.where(onel, v[:, :, None], 0.0), axis=1)
    li_ref[...] = jnp.sum(jnp.where(onel, self_j, 0), axis=1)


def _tc_topk(tmp, ii):
    B, _ = tmp.shape
    return pl.pallas_call(
        _topk_body,
        out_shape=(
            jax.ShapeDtypeStruct((B, BEAM), jnp.float32),
            jax.ShapeDtypeStruct((B, BEAM), jnp.int32),
            jax.ShapeDtypeStruct((B, BEAM), jnp.float32),
            jax.ShapeDtypeStruct((B, BEAM), jnp.int32),
        ),
    )(tmp, ii)


def kernel(weighted_scores, ids):
    ii = ids.astype(jnp.int32)
    tmp = _sc_gather(weighted_scores, ii)
    return _tc_topk(tmp, ii)
